# Initial kernel scaffold; baseline (speedup 1.0000x reference)
import jax
import jax.numpy as jnp
from jax.experimental import pallas as pl


def kernel(x, batch, Ws, bs, Wh, bh, Wo, bo, g1, be1, W1, b1, g2, be2, W2, b2, W3, b3, g3, be3):
    raise NotImplementedError("write your pallas kernel here")


# speedup vs baseline: 12.9429x; 12.9429x over previous
import functools

import jax
import jax.numpy as jnp
from jax import lax
from jax.experimental import pallas as pl
from jax.experimental.pallas import tpu as pltpu

TR = 256
TCOL = 256
PB = 512
KNN = 40
F32_INF_BITS = 0x7F800000


def _p1_body(x_ref, Ws_ref, bs_ref, Wh_ref, bh_ref,
             s_ref, h_ref, sT_ref, hT_ref, sqT_ref):
    x = x_ref[...].astype(jnp.bfloat16)
    Ws = Ws_ref[...].astype(jnp.bfloat16)
    Wh = Wh_ref[...].astype(jnp.bfloat16)
    s = jnp.dot(x, Ws, preferred_element_type=jnp.float32) + bs_ref[...]
    h = jnp.dot(x, Wh, preferred_element_type=jnp.float32) + bh_ref[...]
    s_ref[...] = s
    h_ref[...] = h
    sT_ref[...] = s.T
    hT_ref[...] = h.T
    sq = jnp.sum(s * s, axis=1, keepdims=True)
    sqT_ref[...] = sq.T


def _p2_body(clo_ref, nct_ref,
             s_ref, x_ref, rlo_ref, rhi_ref,
             sT_ref, sqT_ref, h_ref, hT_ref,
             Wox_ref, Woa_ref, bo_ref,
             out_ref, s1_ref, s1q_ref, key_scr, *, n_rows, pd, sd):
    i = pl.program_id(0)
    c_lo = clo_ref[i, 0]
    n_ct = nct_ref[i, 0]
    s_r = s_ref[...]
    sq_r = jnp.sum(s_r * s_r, axis=1, keepdims=True)
    rlo = rlo_ref[...]
    rhi = rhi_ref[...]

    s_r_bf = s_r.astype(jnp.bfloat16)

    def build_tile(t, _):
        c0 = pl.multiple_of(c_lo + t * TCOL, TCOL)
        sT_t = sT_ref[:, pl.ds(c0, TCOL)]
        sq_c = sqT_ref[:, pl.ds(c0, TCOL)]
        dot = lax.dot_general(s_r_bf, sT_t.astype(jnp.bfloat16),
                              (((1,), (0,)), ((), ())),
                              preferred_element_type=jnp.float32)
        d2 = jnp.maximum(sq_r + sq_c - 2.0 * dot, 0.0)
        cols = c0 + lax.broadcasted_iota(jnp.int32, (1, TCOL), 1)
        mask = (cols >= rlo) & (cols < rhi)
        key = jnp.where(mask, lax.bitcast_convert_type(d2, jnp.int32),
                        F32_INF_BITS)
        key_scr[:, pl.ds(pl.multiple_of(t * TCOL, TCOL), TCOL)] = key
        return 0

    lax.fori_loop(0, n_ct, build_tile, 0)

    ones_col = jnp.ones((TCOL, 1), jnp.float32)

    def bs_step(_, lohi):
        lo, hi = lohi
        mid = lo + (hi - lo) // 2

        def count_tile(t, acc):
            key = key_scr[:, pl.ds(pl.multiple_of(t * TCOL, TCOL), TCOL)]
            ind = jnp.where(key <= mid, 1.0, 0.0)
            return acc + jnp.dot(ind, ones_col,
                                 preferred_element_type=jnp.float32)

        cnt = lax.fori_loop(0, n_ct, count_tile,
                            jnp.zeros((TR, 1), jnp.float32))
        ge = cnt >= KNN
        return jnp.where(ge, lo, mid + 1), jnp.where(ge, mid, hi)

    lo0 = jnp.zeros((TR, 1), jnp.int32)
    hi0 = jnp.full((TR, 1), F32_INF_BITS, jnp.int32)
    _, tkey = lax.fori_loop(0, 31, bs_step, (lo0, hi0))

    def agg_tile(t, accs):
        acc_s, acc_m = accs
        c0 = pl.multiple_of(c_lo + t * TCOL, TCOL)
        key = key_scr[:, pl.ds(pl.multiple_of(t * TCOL, TCOL), TCOL)]
        sel = key <= tkey
        terms = []
        for d in range(sd):
            diff = s_r[:, d:d + 1] - sT_ref[d:d + 1, pl.ds(c0, TCOL)]
            terms.append(diff * diff)
        while len(terms) > 1:
            terms = [a + b for a, b in zip(terms[::2], terms[1::2])] + (
                [terms[-1]] if len(terms) % 2 else [])
        d2 = terms[0]
        w = jnp.where(sel, jnp.exp(-10.0 * d2), 0.0)
        h_t = h_ref[pl.ds(c0, TCOL), :]
        acc_s = acc_s + jnp.dot(w, h_t, preferred_element_type=jnp.float32,
                                precision=lax.Precision.HIGHEST)
        mask_bias = jnp.where(sel, 0.0, -jnp.inf)
        cols = []
        for d in range(pd):
            hrow = hT_ref[pl.ds(d, 1), pl.ds(c0, TCOL)]
            cand = w * hrow + mask_bias
            cols.append(jnp.max(cand, axis=1, keepdims=True))
        acc_m = jnp.maximum(acc_m, jnp.concatenate(cols, axis=1))
        return acc_s, acc_m

    acc_s0 = jnp.zeros((TR, pd), jnp.float32)
    acc_m0 = jnp.full((TR, pd), -jnp.inf, jnp.float32)
    acc_s, acc_m = lax.fori_loop(0, n_ct, agg_tile, (acc_s0, acc_m0))

    agg = jnp.concatenate([acc_s * (1.0 / KNN), acc_m], axis=1)
    out = (jnp.dot(x_ref[...].astype(jnp.bfloat16),
                   Wox_ref[...].astype(jnp.bfloat16),
                   preferred_element_type=jnp.float32)
           + jnp.dot(agg.astype(jnp.bfloat16),
                     Woa_ref[...].astype(jnp.bfloat16),
                     preferred_element_type=jnp.float32)
           + bo_ref[...])
    out_ref[...] = out

    rows = i * TR + lax.broadcasted_iota(jnp.int32, (TR, 1), 0)
    valid = rows < n_rows
    outv = jnp.where(valid, out, 0.0)

    @pl.when(i == 0)
    def _():
        s1_ref[...] = jnp.zeros(s1_ref.shape, jnp.float32)
        s1q_ref[...] = jnp.zeros(s1q_ref.shape, jnp.float32)

    s1_ref[...] += jnp.sum(outv, axis=0, keepdims=True)
    s1q_ref[...] += jnp.sum(outv * outv, axis=0, keepdims=True)


def _bn_apply(v, sum_ref, sumsq_ref, g_ref, be_ref, n_rows):
    inv_n = 1.0 / n_rows
    m = sum_ref[...] * inv_n
    var = sumsq_ref[...] * inv_n - m * m
    scale = g_ref[...] / jnp.sqrt(var + 1e-5)
    return (v - m) * scale + be_ref[...]


def _p3_body(out_ref, s1_ref, s1q_ref, g1_ref, be1_ref, W1_ref, b1_ref,
             y1_ref, s2_ref, s2q_ref, *, n_rows):
    i = pl.program_id(0)
    v = _bn_apply(out_ref[...], s1_ref, s1q_ref, g1_ref, be1_ref, n_rows)
    y1 = jnp.tanh(jnp.dot(v.astype(jnp.bfloat16),
                          W1_ref[...].astype(jnp.bfloat16),
                          preferred_element_type=jnp.float32)
                  + b1_ref[...])
    y1_ref[...] = y1
    rows = i * PB + lax.broadcasted_iota(jnp.int32, (PB, 1), 0)
    y1v = jnp.where(rows < n_rows, y1, 0.0)

    @pl.when(i == 0)
    def _():
        s2_ref[...] = jnp.zeros(s2_ref.shape, jnp.float32)
        s2q_ref[...] = jnp.zeros(s2q_ref.shape, jnp.float32)

    s2_ref[...] += jnp.sum(y1v, axis=0, keepdims=True)
    s2q_ref[...] += jnp.sum(y1v * y1v, axis=0, keepdims=True)


def _p4_body(y1_ref, batch_ref, s2_ref, s2q_ref, g2_ref, be2_ref,
             W2_ref, b2_ref,
             y_ref, cnt_ref, esum_ref, emin_ref, emax_ref,
             *, n_rows, nb):
    i = pl.program_id(0)
    v = _bn_apply(y1_ref[...], s2_ref, s2q_ref, g2_ref, be2_ref, n_rows)
    y = jnp.tanh(jnp.dot(v.astype(jnp.bfloat16),
                         W2_ref[...].astype(jnp.bfloat16),
                         preferred_element_type=jnp.float32)
                 + b2_ref[...])
    y_ref[...] = y

    rows = i * PB + lax.broadcasted_iota(jnp.int32, (PB, 1), 0)
    valid = rows < n_rows
    b = batch_ref[...]

    @pl.when(i == 0)
    def _():
        cnt_ref[...] = jnp.zeros(cnt_ref.shape, jnp.float32)
        esum_ref[...] = jnp.zeros(esum_ref.shape, jnp.float32)
        emin_ref[...] = jnp.full(emin_ref.shape, jnp.inf, jnp.float32)
        emax_ref[...] = jnp.full(emax_ref.shape, -jnp.inf, jnp.float32)

    cnts, sums, mins, maxs = [], [], [], []
    for e in range(nb):
        rm = (b == e) & valid
        cnts.append(jnp.sum(rm.astype(jnp.float32), axis=0, keepdims=True))
        sums.append(jnp.sum(jnp.where(rm, y, 0.0), axis=0, keepdims=True))
        mins.append(jnp.min(jnp.where(rm, y, jnp.inf), axis=0, keepdims=True))
        maxs.append(jnp.max(jnp.where(rm, y, -jnp.inf), axis=0, keepdims=True))
    cnt_ref[...] += jnp.concatenate(cnts, axis=0)
    esum_ref[...] += jnp.concatenate(sums, axis=0)
    emin_ref[...] = jnp.minimum(emin_ref[...], jnp.concatenate(mins, axis=0))
    emax_ref[...] = jnp.maximum(emax_ref[...], jnp.concatenate(maxs, axis=0))


def _p5_body(y_ref, batch_ref, cnt_ref, esum_ref, emin_ref, emax_ref,
             W3m_ref, W3y_ref, b3_ref,
             zp_ref, s3_ref, s3q_ref, *, n_rows, nb):
    i = pl.program_id(0)
    cnt = cnt_ref[...]
    nonempty = cnt > 0.0
    mean = esum_ref[...] / jnp.maximum(cnt, 1.0)
    mn = jnp.where(nonempty, emin_ref[...], 0.0)
    mx = jnp.where(nonempty, emax_ref[...], 0.0)
    mmm = jnp.concatenate([mean, mn, mx], axis=1)
    proj = jnp.dot(mmm.astype(jnp.bfloat16),
                   W3m_ref[...].astype(jnp.bfloat16),
                   preferred_element_type=jnp.float32)
    b = batch_ref[...]
    onehot = (b == lax.broadcasted_iota(jnp.int32, (1, nb), 1)
              ).astype(jnp.float32)
    y = y_ref[...]
    zp = jnp.tanh(jnp.dot(onehot, proj,
                          preferred_element_type=jnp.float32,
                          precision=lax.Precision.HIGHEST)
                  + jnp.dot(y.astype(jnp.bfloat16),
                            W3y_ref[...].astype(jnp.bfloat16),
                            preferred_element_type=jnp.float32)
                  + b3_ref[...])
    zp_ref[...] = zp

    rows = i * PB + lax.broadcasted_iota(jnp.int32, (PB, 1), 0)
    zv = jnp.where(rows < n_rows, zp, 0.0)

    @pl.when(i == 0)
    def _():
        s3_ref[...] = jnp.zeros(s3_ref.shape, jnp.float32)
        s3q_ref[...] = jnp.zeros(s3q_ref.shape, jnp.float32)

    s3_ref[...] += jnp.sum(zv, axis=0, keepdims=True)
    s3q_ref[...] += jnp.sum(zv * zv, axis=0, keepdims=True)


def _p6_body(zp_ref, s3_ref, s3q_ref, g3_ref, be3_ref, z_ref, *, n_rows):
    z_ref[...] = _bn_apply(zp_ref[...], s3_ref, s3q_ref, g3_ref, be3_ref,
                           n_rows)


def _cdiv(a, b):
    return (a + b - 1) // b


def kernel(x, batch, Ws, bs, Wh, bh, Wo, bo, g1, be1, W1, b1, g2, be2,
           W2, b2, W3, b3, g3, be3, *, interpret=False):
    n, f_in = x.shape
    sd = Ws.shape[1]
    pd = Wh.shape[1]
    f0 = Wo.shape[1]
    f1 = W1.shape[1]
    f2 = W2.shape[1]
    f3 = W3.shape[1]
    nb = 16
    nf = float(n)

    batch = batch.astype(jnp.int32)
    ev = jnp.arange(nb, dtype=jnp.int32)
    starts = jnp.searchsorted(batch, ev, side="left").astype(jnp.int32)
    ends = jnp.searchsorted(batch, ev, side="right").astype(jnp.int32)
    row_lo = starts[batch].reshape(n, 1)
    row_hi = ends[batch].reshape(n, 1)

    n_chunks = _cdiv(n, TR)
    first = jnp.minimum(jnp.arange(n_chunks, dtype=jnp.int32) * TR, n - 1)
    last = jnp.minimum(first + TR - 1, n - 1)
    c_lo = starts[batch[first]]
    c_hi = ends[batch[last]]
    c_lo_fl = (c_lo // TCOL) * TCOL
    n_ct = jnp.maximum(_cdiv(c_hi - c_lo_fl, TCOL), 1)
    c_lo_fl = c_lo_fl.reshape(n_chunks, 1)
    n_ct = n_ct.reshape(n_chunks, 1)

    f32 = jnp.float32
    s, h, sT, hT, sqT = pl.pallas_call(
        _p1_body,
        grid=(_cdiv(n, PB),),
        in_specs=[
            pl.BlockSpec((PB, f_in), lambda i: (i, 0)),
            pl.BlockSpec((f_in, sd), lambda i: (0, 0)),
            pl.BlockSpec((1, sd), lambda i: (0, 0)),
            pl.BlockSpec((f_in, pd), lambda i: (0, 0)),
            pl.BlockSpec((1, pd), lambda i: (0, 0)),
        ],
        out_specs=[
            pl.BlockSpec((PB, sd), lambda i: (i, 0)),
            pl.BlockSpec((PB, pd), lambda i: (i, 0)),
            pl.BlockSpec((sd, PB), lambda i: (0, i)),
            pl.BlockSpec((pd, PB), lambda i: (0, i)),
            pl.BlockSpec((1, PB), lambda i: (0, i)),
        ],
        out_shape=[
            jax.ShapeDtypeStruct((n, sd), f32),
            jax.ShapeDtypeStruct((n, pd), f32),
            jax.ShapeDtypeStruct((sd, n), f32),
            jax.ShapeDtypeStruct((pd, n), f32),
            jax.ShapeDtypeStruct((1, n), f32),
        ],
        compiler_params=pltpu.CompilerParams(
            dimension_semantics=("arbitrary",)),
        interpret=interpret,
    )(x, Ws, bs.reshape(1, sd), Wh, bh.reshape(1, pd))

    out, s1, s1q = pl.pallas_call(
        functools.partial(_p2_body, n_rows=n, pd=pd, sd=sd),
        grid=(n_chunks,),
        scratch_shapes=[pltpu.VMEM((TR, _cdiv(n, TCOL) * TCOL), jnp.int32)],
        in_specs=[
            pl.BlockSpec(memory_space=pltpu.SMEM),
            pl.BlockSpec(memory_space=pltpu.SMEM),
            pl.BlockSpec((TR, sd), lambda i: (i, 0)),
            pl.BlockSpec((TR, f_in), lambda i: (i, 0)),
            pl.BlockSpec((TR, 1), lambda i: (i, 0)),
            pl.BlockSpec((TR, 1), lambda i: (i, 0)),
            pl.BlockSpec((sd, n), lambda i: (0, 0)),
            pl.BlockSpec((1, n), lambda i: (0, 0)),
            pl.BlockSpec((n, pd), lambda i: (0, 0)),
            pl.BlockSpec((pd, n), lambda i: (0, 0)),
            pl.BlockSpec((f_in, f0), lambda i: (0, 0)),
            pl.BlockSpec((2 * pd, f0), lambda i: (0, 0)),
            pl.BlockSpec((1, f0), lambda i: (0, 0)),
        ],
        out_specs=[
            pl.BlockSpec((TR, f0), lambda i: (i, 0)),
            pl.BlockSpec((1, f0), lambda i: (0, 0)),
            pl.BlockSpec((1, f0), lambda i: (0, 0)),
        ],
        out_shape=[
            jax.ShapeDtypeStruct((n_chunks * TR, f0), f32),
            jax.ShapeDtypeStruct((1, f0), f32),
            jax.ShapeDtypeStruct((1, f0), f32),
        ],
        compiler_params=pltpu.CompilerParams(
            dimension_semantics=("arbitrary",)),
        interpret=interpret,
    )(c_lo_fl, n_ct, s, x, row_lo, row_hi, sT, sqT, h, hT,
      Wo[:f_in], Wo[f_in:], bo.reshape(1, f0))
    out = out[:n]

    npb = _cdiv(n, PB)
    y1, s2, s2q = pl.pallas_call(
        functools.partial(_p3_body, n_rows=nf),
        grid=(npb,),
        in_specs=[
            pl.BlockSpec((PB, f0), lambda i: (i, 0)),
            pl.BlockSpec((1, f0), lambda i: (0, 0)),
            pl.BlockSpec((1, f0), lambda i: (0, 0)),
            pl.BlockSpec((1, f0), lambda i: (0, 0)),
            pl.BlockSpec((1, f0), lambda i: (0, 0)),
            pl.BlockSpec((f0, f1), lambda i: (0, 0)),
            pl.BlockSpec((1, f1), lambda i: (0, 0)),
        ],
        out_specs=[
            pl.BlockSpec((PB, f1), lambda i: (i, 0)),
            pl.BlockSpec((1, f1), lambda i: (0, 0)),
            pl.BlockSpec((1, f1), lambda i: (0, 0)),
        ],
        out_shape=[
            jax.ShapeDtypeStruct((n, f1), f32),
            jax.ShapeDtypeStruct((1, f1), f32),
            jax.ShapeDtypeStruct((1, f1), f32),
        ],
        compiler_params=pltpu.CompilerParams(
            dimension_semantics=("arbitrary",)),
        interpret=interpret,
    )(out, s1, s1q, g1.reshape(1, f0), be1.reshape(1, f0), W1,
      b1.reshape(1, f1))

    batch2d = batch.reshape(n, 1)
    y, cnt, esum, emin, emax = pl.pallas_call(
        functools.partial(_p4_body, n_rows=nf, nb=nb),
        grid=(npb,),
        in_specs=[
            pl.BlockSpec((PB, f1), lambda i: (i, 0)),
            pl.BlockSpec((PB, 1), lambda i: (i, 0)),
            pl.BlockSpec((1, f1), lambda i: (0, 0)),
            pl.BlockSpec((1, f1), lambda i: (0, 0)),
            pl.BlockSpec((1, f1), lambda i: (0, 0)),
            pl.BlockSpec((1, f1), lambda i: (0, 0)),
            pl.BlockSpec((f1, f2), lambda i: (0, 0)),
            pl.BlockSpec((1, f2), lambda i: (0, 0)),
        ],
        out_specs=[
            pl.BlockSpec((PB, f2), lambda i: (i, 0)),
            pl.BlockSpec((nb, 1), lambda i: (0, 0)),
            pl.BlockSpec((nb, f2), lambda i: (0, 0)),
            pl.BlockSpec((nb, f2), lambda i: (0, 0)),
            pl.BlockSpec((nb, f2), lambda i: (0, 0)),
        ],
        out_shape=[
            jax.ShapeDtypeStruct((n, f2), f32),
            jax.ShapeDtypeStruct((nb, 1), f32),
            jax.ShapeDtypeStruct((nb, f2), f32),
            jax.ShapeDtypeStruct((nb, f2), f32),
            jax.ShapeDtypeStruct((nb, f2), f32),
        ],
        compiler_params=pltpu.CompilerParams(
            dimension_semantics=("arbitrary",)),
        interpret=interpret,
    )(y1, batch2d, s2, s2q, g2.reshape(1, f1), be2.reshape(1, f1), W2,
      b2.reshape(1, f2))

    zp, s3, s3q = pl.pallas_call(
        functools.partial(_p5_body, n_rows=nf, nb=nb),
        grid=(npb,),
        in_specs=[
            pl.BlockSpec((PB, f2), lambda i: (i, 0)),
            pl.BlockSpec((PB, 1), lambda i: (i, 0)),
            pl.BlockSpec((nb, 1), lambda i: (0, 0)),
            pl.BlockSpec((nb, f2), lambda i: (0, 0)),
            pl.BlockSpec((nb, f2), lambda i: (0, 0)),
            pl.BlockSpec((nb, f2), lambda i: (0, 0)),
            pl.BlockSpec((3 * f2, f3), lambda i: (0, 0)),
            pl.BlockSpec((f2, f3), lambda i: (0, 0)),
            pl.BlockSpec((1, f3), lambda i: (0, 0)),
        ],
        out_specs=[
            pl.BlockSpec((PB, f3), lambda i: (i, 0)),
            pl.BlockSpec((1, f3), lambda i: (0, 0)),
            pl.BlockSpec((1, f3), lambda i: (0, 0)),
        ],
        out_shape=[
            jax.ShapeDtypeStruct((n, f3), f32),
            jax.ShapeDtypeStruct((1, f3), f32),
            jax.ShapeDtypeStruct((1, f3), f32),
        ],
        compiler_params=pltpu.CompilerParams(
            dimension_semantics=("arbitrary",)),
        interpret=interpret,
    )(y, batch2d, cnt, esum, emin, emax, W3[:3 * f2], W3[3 * f2:],
      b3.reshape(1, f3))

    z = pl.pallas_call(
        functools.partial(_p6_body, n_rows=nf),
        grid=(npb,),
        in_specs=[
            pl.BlockSpec((PB, f3), lambda i: (i, 0)),
            pl.BlockSpec((1, f3), lambda i: (0, 0)),
            pl.BlockSpec((1, f3), lambda i: (0, 0)),
            pl.BlockSpec((1, f3), lambda i: (0, 0)),
            pl.BlockSpec((1, f3), lambda i: (0, 0)),
        ],
        out_specs=pl.BlockSpec((PB, f3), lambda i: (i, 0)),
        out_shape=jax.ShapeDtypeStruct((n, f3), f32),
        compiler_params=pltpu.CompilerParams(
            dimension_semantics=("arbitrary",)),
        interpret=interpret,
    )(zp, s3, s3q, g3.reshape(1, f3), be3.reshape(1, f3))
    return z
